# Initial kernel scaffold; baseline (speedup 1.0000x reference)
import jax
import jax.numpy as jnp
from jax.experimental import pallas as pl


def kernel(x, embedding):
    raise NotImplementedError("write your pallas kernel here")


# speedup vs baseline: 1.6520x; 1.6520x over previous
import functools

import jax
import jax.numpy as jnp
from jax import lax
from jax.experimental import pallas as pl
from jax.experimental.pallas import tpu as pltpu
from jax.experimental.pallas import tpu_sc as plsc

B, N, D = 4, 8192, 1024

_info = plsc.get_sparse_core_info()
NC, NS = _info.num_cores, _info.num_subcores
NW = NC * NS
ROWS_PER_W = N // NW
CHUNK = 64
NCHUNK = ROWS_PER_W // CHUNK

_mesh = plsc.VectorSubcoreMesh(core_axis_name="c", subcore_axis_name="s")


@functools.partial(
    pl.kernel,
    mesh=_mesh,
    out_type=jax.ShapeDtypeStruct((B, N, D), jnp.float32),
    scratch_types=[pltpu.VMEM((CHUNK, D), jnp.float32)],
)
def _broadcast_rows(emb_hbm, out_hbm, buf):
    wid = lax.axis_index("s") * NC + lax.axis_index("c")
    base = wid * ROWS_PER_W
    for ci in range(NCHUNK):
        r0 = base + ci * CHUNK
        pltpu.sync_copy(emb_hbm.at[pl.ds(r0, CHUNK)], buf)
        for b in range(B):
            pltpu.sync_copy(buf, out_hbm.at[b, pl.ds(r0, CHUNK)])


def kernel(x, embedding):
    del x
    return _broadcast_rows(embedding)
